# baseline (speedup 1.0000x reference)
import functools

import jax
import jax.numpy as jnp
from jax import lax
from jax.experimental import pallas as pl
from jax.experimental.pallas import tpu as pltpu
from jax.experimental.pallas import tpu_sc as plsc

N, D = 16384, 64

_info = plsc.get_sparse_core_info()
NC, NS, L = _info.num_cores, _info.num_subcores, _info.num_lanes
NW = NC * NS
ROWS = N // NW
CH = 4
CR = ROWS // CH
PW = L + 1

_mesh = plsc.VectorSubcoreMesh(core_axis_name="c", subcore_axis_name="s")


@functools.partial(
    pl.kernel,
    out_type=jax.ShapeDtypeStruct((N,), jnp.float32),
    mesh=_mesh,
    compiler_params=pltpu.CompilerParams(needs_layout_passes=False),
    scratch_types=[
        pltpu.VMEM((CR, D), jnp.float32),
        pltpu.VMEM((CR, D), jnp.float32),
        pltpu.VMEM((CR, D), jnp.float32),
        pltpu.VMEM((CR, D), jnp.float32),
        pltpu.VMEM((CR, PW), jnp.float32),
        pltpu.VMEM((ROWS,), jnp.float32),
        pltpu.SemaphoreType.DMA,
        pltpu.SemaphoreType.DMA,
    ],
)
def _rowdot(gu_hbm, gi_hbm, out_hbm, u0, i0, u1, i1, p_v, o_v, sem0, sem1):
    wid = lax.axis_index("s") * NC + lax.axis_index("c")
    base = wid * ROWS
    lanes = lax.iota(jnp.int32, L)
    bufs = ((u0, i0, sem0), (u1, i1, sem1))

    def start(c, buf):
        u, i, sem = buf
        cu = pltpu.async_copy(gu_hbm.at[pl.ds(base + c * CR, CR), :], u, sem)
        ci = pltpu.async_copy(gi_hbm.at[pl.ds(base + c * CR, CR), :], i, sem)
        return cu, ci

    pending = start(0, bufs[0])
    for c in range(CH):
        nxt = start(c + 1, bufs[(c + 1) % 2]) if c + 1 < CH else None
        pending[0].wait()
        pending[1].wait()
        pending = nxt
        u, i, _ = bufs[c % 2]

        def grp_body(g, carry, u=u, i=i, c=c):
            rb = g * L
            for l in range(L):
                r = rb + l
                acc = u[r, pl.ds(0, L)] * i[r, pl.ds(0, L)]
                for j in range(1, D // L):
                    acc = acc + u[r, pl.ds(j * L, L)] * i[r, pl.ds(j * L, L)]
                p_v[r, pl.ds(0, L)] = acc
            rows = rb + lanes
            cols = [plsc.load_gather(p_v, [rows, jnp.full((L,), j, jnp.int32)])
                    for j in range(L)]
            while len(cols) > 1:
                cols = [cols[k] + cols[k + 1] for k in range(0, len(cols), 2)]
            o_v[pl.ds(c * CR + g * L, L)] = cols[0]
            return carry

        lax.fori_loop(0, CR // L, grp_body, 0)

    pltpu.sync_copy(o_v, out_hbm.at[pl.ds(base, ROWS)])


def kernel(gu, gi):
    return _rowdot(gu, gi)
